# baseline (speedup 1.0000x reference)
import functools

import jax
import jax.numpy as jnp
import numpy as np
from jax import lax
from jax.experimental import pallas as pl
from jax.experimental.pallas import tpu as pltpu
from jax.experimental.pallas import tpu_sc as plsc

NUM_NEGATIVES = 100
TEMPERATURE = 0.05

_D = 64
_K = 112
_KG = 7

_SIGMA = np.array([0, 8, 4, 12, 2, 10, 6, 14, 1, 9, 5, 13, 3, 11, 7, 15])
_PERM_SRC = np.zeros(_K, dtype=np.int32)
for _g in range(_KG):
    _PERM_SRC[16 * _g + _SIGMA] = 16 * _g + np.arange(16)


def _normalize_rows_body(x_ref, o_ref):
    x = x_ref[...]
    n = jnp.sqrt(jnp.sum(x * x, axis=1, keepdims=True))
    o_ref[...] = x / jnp.maximum(n, 1e-12)


def _normalize_rows(x, block_rows):
    rows, d = x.shape
    grid = (rows + block_rows - 1) // block_rows
    return pl.pallas_call(
        _normalize_rows_body,
        grid=(grid,),
        in_specs=[pl.BlockSpec((block_rows, d), lambda i: (i, 0))],
        out_specs=pl.BlockSpec((block_rows, d), lambda i: (i, 0)),
        out_shape=jax.ShapeDtypeStruct((rows, d), x.dtype),
    )(x)


_CH = 4


@functools.lru_cache(maxsize=None)
def _make_sc_call(n_tok):
    mesh = plsc.VectorSubcoreMesh(core_axis_name="c", subcore_axis_name="s")
    nc, ns = mesh.num_cores, mesh.num_subcores
    nw = nc * ns
    ntok_w = n_tok // nw
    nch = ntok_w // _CH

    def body(table, idx, vtab, out, idx_v, v_v, rows_v, out_v,
             sg0, sg1, ss0, ss1):
        wid = lax.axis_index("s") * nc + lax.axis_index("c")
        base = pl.multiple_of(wid * ntok_w, 8)

        iot = lax.iota(jnp.int32, 16)
        masks = {d: (iot & d) == 0 for d in (8, 4, 2, 1)}
        perms = {d: jnp.bitwise_xor(iot, d) for d in (8, 4, 2, 1)}
        mask_last = (iot < (NUM_NEGATIVES + 1 - 16 * (_KG - 1))).astype(
            jnp.float32
        )
        ones16 = jnp.ones((16,), jnp.float32)
        zeros16 = jnp.zeros((16,), jnp.float32)

        _gdn = lax.GatherDimensionNumbers(
            offset_dims=(), collapsed_slice_dims=(0,), start_index_map=(0,)
        )

        def take16(v, idxvec):
            return lax.gather(
                v, idxvec[:, None], _gdn, (1,),
                mode=lax.GatherScatterMode.PROMISE_IN_BOUNDS,
            )

        def fold(a, b, d):
            sel_a = jnp.where(masks[d], a, b)
            sel_b = jnp.where(masks[d], b, a)
            return sel_a + take16(sel_b, perms[d])

        def lane_sum_16(ps):
            cur = list(ps)
            for d in (8, 4, 2, 1):
                cur = [fold(cur[2 * i], cur[2 * i + 1], d)
                       for i in range(len(cur) // 2)]
            return cur[0]

        sgs = (sg0, sg1)
        sss = (ss0, ss1)

        def stage_pair(cp, P):
            t0 = pl.multiple_of(base + cp * 8, 8)
            pltpu.async_copy(idx.at[pl.ds(t0, 8)], idx_v.at[P], sss[P])
            pltpu.async_copy(vtab.at[pl.ds(t0, 8)], v_v.at[P], sss[P])

        def stage_wait(P):
            pltpu.make_async_copy(
                idx.at[pl.ds(base, 8)], idx_v.at[P], sss[P]
            ).wait()
            pltpu.make_async_copy(
                vtab.at[pl.ds(base, 8)], v_v.at[P], sss[P]
            ).wait()

        def fire(P, hh, q):
            for u in range(_CH):
                pltpu.async_copy(
                    table.at[idx_v.at[P, hh * _CH + u]], rows_v.at[q, u],
                    sgs[q],
                )

        def drain(q):
            for u in range(_CH):
                pltpu.make_async_copy(
                    table.at[idx_v.at[0, 0]], rows_v.at[q, 0], sgs[q]
                ).wait()

        def compute_half(e, P, hh, q):
            def ubody(u, carry):
                vv = [v_v[P, hh * _CH + u, pl.ds(cc * 16, 16)]
                      for cc in range(4)]

                def gbody(g, gc):
                    ssum, logits0 = gc
                    ps = []
                    for i in range(16):
                        r = g * 16 + i
                        p0 = rows_v[q, u, r, pl.ds(0, 16)] * vv[0]
                        p1 = rows_v[q, u, r, pl.ds(16, 16)] * vv[1]
                        p2 = rows_v[q, u, r, pl.ds(32, 16)] * vv[2]
                        p3 = rows_v[q, u, r, pl.ds(48, 16)] * vv[3]
                        ps.append((p0 + p1) + (p2 + p3))
                    lg = lane_sum_16(ps) * (1.0 / TEMPERATURE)
                    term = jnp.exp(lg)
                    mvec = jnp.where(
                        jnp.full((16,), g == _KG - 1), mask_last, ones16
                    )
                    ssum = ssum + term * mvec
                    logits0 = jnp.where(jnp.full((16,), g == 0), lg, logits0)
                    return (ssum, logits0)

                ssum, logits0 = lax.fori_loop(
                    0, _KG, gbody, (zeros16, zeros16)
                )
                out_v[e * _CH + u, pl.ds(0, 16)] = ssum
                out_v[e * _CH + u, pl.ds(16, 16)] = logits0
                return carry

            lax.fori_loop(0, _CH, ubody, 0)

        stage_pair(0, 0)
        stage_wait(0)
        fire(0, 0, 0)
        stage_pair(1, 1)
        nblk = ntok_w // (4 * _CH)

        def blk_body(mm, carry):
            fire(0, 1, 1)
            drain(0)
            compute_half(0, 0, 0, 0)

            stage_wait(1)
            fire(1, 0, 0)
            drain(1)
            compute_half(1, 0, 1, 1)

            @pl.when(mm + 1 < nblk)
            def _():
                stage_pair(2 * mm + 2, 0)

            fire(1, 1, 1)
            drain(0)
            compute_half(2, 1, 0, 0)

            @pl.when(mm + 1 < nblk)
            def _():
                stage_wait(0)
                fire(0, 0, 0)

            drain(1)
            compute_half(3, 1, 1, 1)

            @pl.when(mm + 1 < nblk)
            def _():
                stage_pair(2 * mm + 3, 1)

            t0 = pl.multiple_of(base + mm * 4 * _CH, 8)
            pltpu.sync_copy(out_v, out.at[pl.ds(t0, 4 * _CH)])
            return carry

        lax.fori_loop(0, nblk, blk_body, 0)

    return pl.kernel(
        body,
        out_type=jax.ShapeDtypeStruct((n_tok, 32), jnp.float32),
        mesh=mesh,
        compiler_params=pltpu.CompilerParams(
            needs_layout_passes=False, use_tc_tiling_on_sc=False
        ),
        scratch_types=[
            pltpu.VMEM((2, 2 * _CH, _K), jnp.int32),
            pltpu.VMEM((2, 2 * _CH, _D), jnp.float32),
            pltpu.VMEM((2, _CH, _K, _D), jnp.float32),
            pltpu.VMEM((4 * _CH, 32), jnp.float32),
            pltpu.SemaphoreType.DMA,
            pltpu.SemaphoreType.DMA,
            pltpu.SemaphoreType.DMA,
            pltpu.SemaphoreType.DMA,
        ],
    )


def _final_body(s_ref, w_ref, o_ref):
    s = s_ref[...]
    w = w_ref[...]
    ssum = jnp.sum(s[:, 0:16], axis=1, keepdims=True)
    loss = jnp.log(ssum) - s[:, 16:17]
    wcol = w[:, 0:1]
    num = jnp.sum(loss * wcol)
    den = jnp.sum(wcol)
    o_ref[...] = jnp.reshape(num / den, (1, 1))


def _final_call(sc_out, w32):
    return pl.pallas_call(
        _final_body,
        out_shape=jax.ShapeDtypeStruct((1, 1), jnp.float32),
    )(sc_out, w32)


def kernel(output_embeddings, target_ids, all_item_embeddings, supervision_weights):
    b, s, d = output_embeddings.shape
    n = b * s
    num_items = all_item_embeddings.shape[0] - 1

    flat_output = output_embeddings.reshape(-1, d)
    flat_targets = target_ids.reshape(-1)
    flat_weights = supervision_weights.reshape(-1)

    nk = jax.random.key(12345)
    nk1, nk2 = jax.random.split(nk)
    neg = jax.random.randint(nk1, (n, NUM_NEGATIVES), 1, num_items + 1)
    res = jax.random.randint(nk2, (n, NUM_NEGATIVES), 1, num_items + 1)
    neg = jnp.where(neg != flat_targets[:, None], neg, res)
    neg_idx = jnp.clip(neg - 1, 0, num_items)
    tgt_idx = jnp.clip(flat_targets - 1, 0, num_items)
    pad = jnp.zeros((n, _K - 1 - NUM_NEGATIVES), jnp.int32)
    cols = jnp.concatenate(
        [tgt_idx[:, None], neg_idx, pad], axis=1
    ).astype(jnp.int32)
    idx_all = cols[:, _PERM_SRC]

    norm_table = _normalize_rows(all_item_embeddings, 1024)
    vnorm = _normalize_rows(flat_output, 2048)

    sc_out = _make_sc_call(n)(norm_table, idx_all, vnorm)

    w32 = jnp.broadcast_to(flat_weights[:, None], (n, 32))
    return _final_call(sc_out, w32)[0, 0]


# speedup vs baseline: 7.9239x; 7.9239x over previous
import functools

import jax
import jax.numpy as jnp
import numpy as np
from jax import lax
from jax.experimental import pallas as pl
from jax.experimental.pallas import tpu as pltpu
from jax.experimental.pallas import tpu_sc as plsc

NUM_NEGATIVES = 100
TEMPERATURE = 0.05

_D = 64
_K = 104
_KG = 7

_SIGMA = np.array([0, 8, 4, 12, 2, 10, 6, 14, 1, 9, 5, 13, 3, 11, 7, 15])
_PERM_SRC = np.zeros(_K, dtype=np.int32)
for _g in range(6):
    _PERM_SRC[16 * _g + _SIGMA] = 16 * _g + np.arange(16)
_G6_POS = [i for i in range(16) if 96 + _SIGMA[i] <= 100]
_G6_ROW = {i: 96 + j for j, i in enumerate(_G6_POS)}
for _j, _i in enumerate(_G6_POS):
    _PERM_SRC[96 + _j] = 96 + _SIGMA[_i]
_PERM_SRC[101:104] = (101, 102, 103)


def _normalize_rows_body(x_ref, o_ref):
    x = x_ref[...]
    n = jnp.sqrt(jnp.sum(x * x, axis=1, keepdims=True))
    o_ref[...] = x / jnp.maximum(n, 1e-12)


def _normalize_rows(x, block_rows):
    rows, d = x.shape
    grid = (rows + block_rows - 1) // block_rows
    return pl.pallas_call(
        _normalize_rows_body,
        grid=(grid,),
        in_specs=[pl.BlockSpec((block_rows, d), lambda i: (i, 0))],
        out_specs=pl.BlockSpec((block_rows, d), lambda i: (i, 0)),
        out_shape=jax.ShapeDtypeStruct((rows, d), x.dtype),
    )(x)


_CH = 4


@functools.lru_cache(maxsize=None)
def _make_sc_call(n_tok):
    mesh = plsc.VectorSubcoreMesh(core_axis_name="c", subcore_axis_name="s")
    nc, ns = mesh.num_cores, mesh.num_subcores
    nw = nc * ns
    ntok_w = n_tok // nw
    nch = ntok_w // _CH

    def body(table, idx, vtab, out, idx_v, v_v, rows_v, out_v,
             sg0, sg1, ss0, ss1):
        wid = lax.axis_index("s") * nc + lax.axis_index("c")
        base = pl.multiple_of(wid * ntok_w, 8)

        iot = lax.iota(jnp.int32, 16)
        masks = {d: (iot & d) == 0 for d in (8, 4, 2, 1)}
        perms = {d: jnp.bitwise_xor(iot, d) for d in (8, 4, 2, 1)}
        mask_last = (iot < (NUM_NEGATIVES + 1 - 16 * (_KG - 1))).astype(
            jnp.float32
        )
        ones16 = jnp.ones((16,), jnp.float32)
        zeros16 = jnp.zeros((16,), jnp.float32)

        _gdn = lax.GatherDimensionNumbers(
            offset_dims=(), collapsed_slice_dims=(0,), start_index_map=(0,)
        )

        def take16(v, idxvec):
            return lax.gather(
                v, idxvec[:, None], _gdn, (1,),
                mode=lax.GatherScatterMode.PROMISE_IN_BOUNDS,
            )

        def fold(a, b, d):
            sel_a = jnp.where(masks[d], a, b)
            sel_b = jnp.where(masks[d], b, a)
            return sel_a + take16(sel_b, perms[d])

        def lane_sum_16(ps):
            cur = list(ps)
            for d in (8, 4, 2, 1):
                cur = [fold(cur[2 * i], cur[2 * i + 1], d)
                       for i in range(len(cur) // 2)]
            return cur[0]

        sgs = (sg0, sg1)
        sss = (ss0, ss1)

        def stage_pair(cp, P):
            t0 = pl.multiple_of(base + cp * 8, 8)
            pltpu.async_copy(idx.at[pl.ds(t0, 8)], idx_v.at[P], sss[P])
            pltpu.async_copy(vtab.at[pl.ds(t0, 8)], v_v.at[P], sss[P])

        def stage_wait(P):
            pltpu.make_async_copy(
                idx.at[pl.ds(base, 8)], idx_v.at[P], sss[P]
            ).wait()
            pltpu.make_async_copy(
                vtab.at[pl.ds(base, 8)], v_v.at[P], sss[P]
            ).wait()

        def fire(P, hh, q):
            for u in range(_CH):
                pltpu.async_copy(
                    table.at[idx_v.at[P, hh * _CH + u]], rows_v.at[q, u],
                    sgs[q],
                )

        def drain(q):
            for u in range(_CH):
                pltpu.make_async_copy(
                    table.at[idx_v.at[0, 0]], rows_v.at[q, 0], sgs[q]
                ).wait()

        def compute_half(e, P, hh, q):
            def ubody(u, carry):
                vv = [v_v[P, hh * _CH + u, pl.ds(cc * 16, 16)]
                      for cc in range(4)]

                def dot_row(r):
                    p0 = rows_v[q, u, r, pl.ds(0, 16)] * vv[0]
                    p1 = rows_v[q, u, r, pl.ds(16, 16)] * vv[1]
                    p2 = rows_v[q, u, r, pl.ds(32, 16)] * vv[2]
                    p3 = rows_v[q, u, r, pl.ds(48, 16)] * vv[3]
                    return (p0 + p1) + (p2 + p3)

                def gbody(g, gc):
                    ssum, logits0 = gc
                    ps = [dot_row(g * 16 + i) for i in range(16)]
                    lg = lane_sum_16(ps) * (1.0 / TEMPERATURE)
                    ssum = ssum + jnp.exp(lg)
                    logits0 = jnp.where(jnp.full((16,), g == 0), lg, logits0)
                    return (ssum, logits0)

                ssum, logits0 = lax.fori_loop(
                    0, _KG - 1, gbody, (zeros16, zeros16)
                )
                ps6 = [dot_row(_G6_ROW[i]) if i in _G6_ROW else zeros16
                       for i in range(16)]
                lg6 = lane_sum_16(ps6) * (1.0 / TEMPERATURE)
                ssum = ssum + jnp.exp(lg6) * mask_last
                out_v[e * _CH + u, pl.ds(0, 16)] = ssum
                out_v[e * _CH + u, pl.ds(16, 16)] = logits0
                return carry

            lax.fori_loop(0, _CH, ubody, 0)

        stage_pair(0, 0)
        stage_wait(0)
        fire(0, 0, 0)
        stage_pair(1, 1)
        nblk = ntok_w // (4 * _CH)

        def blk_body(mm, carry):
            fire(0, 1, 1)
            drain(0)
            compute_half(0, 0, 0, 0)

            stage_wait(1)
            fire(1, 0, 0)
            drain(1)
            compute_half(1, 0, 1, 1)

            @pl.when(mm + 1 < nblk)
            def _():
                stage_pair(2 * mm + 2, 0)

            fire(1, 1, 1)
            drain(0)
            compute_half(2, 1, 0, 0)

            @pl.when(mm + 1 < nblk)
            def _():
                stage_wait(0)
                fire(0, 0, 0)

            drain(1)
            compute_half(3, 1, 1, 1)

            @pl.when(mm + 1 < nblk)
            def _():
                stage_pair(2 * mm + 3, 1)

            t0 = pl.multiple_of(base + mm * 4 * _CH, 8)
            pltpu.sync_copy(out_v, out.at[pl.ds(t0, 4 * _CH)])
            return carry

        lax.fori_loop(0, nblk, blk_body, 0)

    return pl.kernel(
        body,
        out_type=jax.ShapeDtypeStruct((n_tok, 32), jnp.float32),
        mesh=mesh,
        compiler_params=pltpu.CompilerParams(
            needs_layout_passes=False, use_tc_tiling_on_sc=False
        ),
        scratch_types=[
            pltpu.VMEM((2, 2 * _CH, _K), jnp.int32),
            pltpu.VMEM((2, 2 * _CH, _D), jnp.float32),
            pltpu.VMEM((2, _CH, _K, _D), jnp.float32),
            pltpu.VMEM((4 * _CH, 32), jnp.float32),
            pltpu.SemaphoreType.DMA,
            pltpu.SemaphoreType.DMA,
            pltpu.SemaphoreType.DMA,
            pltpu.SemaphoreType.DMA,
        ],
    )


def _final_body(s_ref, w_ref, o_ref):
    s = s_ref[...]
    w = w_ref[...]
    ssum = jnp.sum(s[:, 0:16], axis=1, keepdims=True)
    loss = jnp.log(ssum) - s[:, 16:17]
    wcol = w[:, 0:1]
    num = jnp.sum(loss * wcol)
    den = jnp.sum(wcol)
    o_ref[...] = jnp.reshape(num / den, (1, 1))


def _final_call(sc_out, w32):
    return pl.pallas_call(
        _final_body,
        out_shape=jax.ShapeDtypeStruct((1, 1), jnp.float32),
    )(sc_out, w32)


def kernel(output_embeddings, target_ids, all_item_embeddings, supervision_weights):
    b, s, d = output_embeddings.shape
    n = b * s
    num_items = all_item_embeddings.shape[0] - 1

    flat_output = output_embeddings.reshape(-1, d)
    flat_targets = target_ids.reshape(-1)
    flat_weights = supervision_weights.reshape(-1)

    nk = jax.random.key(12345)
    nk1, nk2 = jax.random.split(nk)
    neg = jax.random.randint(nk1, (n, NUM_NEGATIVES), 1, num_items + 1)
    res = jax.random.randint(nk2, (n, NUM_NEGATIVES), 1, num_items + 1)
    neg = jnp.where(neg != flat_targets[:, None], neg, res)
    neg_idx = jnp.clip(neg - 1, 0, num_items)
    tgt_idx = jnp.clip(flat_targets - 1, 0, num_items)
    npad = _K - 1 - NUM_NEGATIVES
    pad = (
        jnp.arange(n, dtype=jnp.int32)[:, None]
        + jnp.arange(npad, dtype=jnp.int32)[None, :] * 7919
    ) % num_items
    cols = jnp.concatenate(
        [tgt_idx[:, None], neg_idx, pad], axis=1
    ).astype(jnp.int32)
    idx_all = cols[:, _PERM_SRC]

    norm_table = _normalize_rows(all_item_embeddings, 1024)
    vnorm = _normalize_rows(flat_output, 2048)

    sc_out = _make_sc_call(n)(norm_table, idx_all, vnorm)

    w32 = jnp.broadcast_to(flat_weights[:, None], (n, 32))
    return _final_call(sc_out, w32)[0, 0]
